# baseline (speedup 1.0000x reference)
import dataclasses
import functools

import jax
import jax.numpy as jnp
from jax import lax
from jax.experimental import pallas as pl
from jax.experimental.pallas import tpu as pltpu
from jax.experimental.pallas import tpu_sc as plsc

N_EMBED = 16
BATCH = 16384
NUM_CORES = 2
NUM_SUBCORES = 16
NUM_WORKERS = NUM_CORES * NUM_SUBCORES
B_PER_W = BATCH // NUM_WORKERS
LANES = 16


def _sc_embed_dot(u_idx, i_idx, mu, bias, W_user, W_item):
    mesh = plsc.VectorSubcoreMesh(core_axis_name="c", subcore_axis_name="s")

    cp = pltpu.CompilerParams()
    if "needs_layout_passes" in pltpu.CompilerParams.__dataclass_fields__:
        cp = dataclasses.replace(cp, needs_layout_passes=False)
    if "use_tc_tiling_on_sc" in pltpu.CompilerParams.__dataclass_fields__:
        cp = dataclasses.replace(cp, use_tc_tiling_on_sc=False)

    @functools.partial(
        pl.kernel,
        compiler_params=cp,
        out_type=jax.ShapeDtypeStruct((BATCH,), jnp.float32),
        mesh=mesh,
        scratch_types=[
            pltpu.VMEM((B_PER_W,), jnp.int32),
            pltpu.VMEM((B_PER_W,), jnp.int32),
            pltpu.VMEM((B_PER_W, N_EMBED), jnp.float32),
            pltpu.VMEM((B_PER_W, N_EMBED), jnp.float32),
            pltpu.VMEM((B_PER_W,), jnp.float32),
            pltpu.VMEM((B_PER_W,), jnp.float32),
            pltpu.VMEM((B_PER_W,), jnp.float32),
            pltpu.SemaphoreType.DMA,
        ],
    )
    def k(u_hbm, i_hbm, mu_hbm, b_hbm, wu_hbm, wi_hbm, out_hbm,
          uidx_v, iidx_v, urows_v, irows_v, mu_v, b_v, out_v, sem):
        wid = lax.axis_index("s") * NUM_CORES + lax.axis_index("c")
        base = wid * B_PER_W
        sl = pl.ds(base, B_PER_W)

        pltpu.sync_copy(u_hbm.at[sl], uidx_v)
        pltpu.sync_copy(i_hbm.at[sl], iidx_v)
        cp_u = pltpu.async_copy(wu_hbm.at[uidx_v], urows_v, sem)
        cp_i = pltpu.async_copy(wi_hbm.at[iidx_v], irows_v, sem)
        pltpu.sync_copy(mu_hbm.at[sl], mu_v)
        pltpu.sync_copy(b_hbm.at[sl], b_v)
        cp_u.wait()
        cp_i.wait()

        lane_iota = lax.iota(jnp.int32, LANES)

        @pl.loop(0, B_PER_W, step=LANES)
        def _(c):
            rows = lane_iota + c
            acc = mu_v[pl.ds(c, LANES)] + 2.0 * b_v[pl.ds(c, LANES)]
            for e in range(N_EMBED):
                col = jnp.full((LANES,), e, jnp.int32)
                uv = plsc.load_gather(urows_v, [rows, col])
                iv = plsc.load_gather(irows_v, [rows, col])
                acc = acc + uv * iv
            out_v[pl.ds(c, LANES)] = acc

        pltpu.sync_copy(out_v, out_hbm.at[sl])

    return k(u_idx, i_idx, mu, bias, W_user, W_item)


def kernel(x, W_user, W_item):
    u_idx = x[:, 0].astype(jnp.int32)
    i_idx = x[:, 1].astype(jnp.int32)
    mu = x[:, 2]
    bias = x[:, 3]
    return _sc_embed_dot(u_idx, i_idx, mu, bias, W_user, W_item)
